# baseline (speedup 1.0000x reference)
import functools
import jax
import jax.numpy as jnp
from jax import lax
from jax.experimental import pallas as pl
from jax.experimental.pallas import tpu as pltpu
from jax.experimental.pallas import tpu_sc as plsc

N = 50000
NODE_BLK = 2048
GRID = 25
NPAD = NODE_BLK * GRID
DUMMY = N
E = 800000
EPAD = 819200
EROWS = EPAD // 128
NSUB = 16
SUB_ROWS = NPAD // NSUB

_mesh = plsc.VectorSubcoreMesh(core_axis_name="c", subcore_axis_name="s")
_sc_params = pltpu.CompilerParams(use_tc_tiling_on_sc=False)


@functools.partial(
    pl.kernel,
    out_type=[
        jax.ShapeDtypeStruct((EROWS, 128), jnp.int32),
        jax.ShapeDtypeStruct((2, NPAD), jnp.float32),
    ],
    mesh=_mesh,
    scratch_types=[
        pltpu.VMEM((4, 128), jnp.int32),
        pltpu.VMEM((4, 128), jnp.int32),
        pltpu.VMEM((4, 128), jnp.int32),
        pltpu.VMEM((4, 128), jnp.float32),
        pltpu.VMEM((SUB_ROWS,), jnp.float32),
        pltpu.VMEM_SHARED((NPAD,), jnp.float32),
    ],
    compiler_params=_sc_params,
)
def _prep(rows_hbm, cols_hbm, colp_hbm, deg_hbm, rbuf, cbuf, cpbuf, ewbuf,
          zbuf, deg_acc):
    c = lax.axis_index("c")
    s = lax.axis_index("s")

    def zb(k, carry):
        zbuf[pl.ds(k * 16, 16)] = jnp.zeros((16,), jnp.float32)
        return carry

    lax.fori_loop(0, SUB_ROWS // 16, zb, None)
    pltpu.sync_copy(zbuf, deg_acc.at[pl.ds(s * SUB_ROWS, SUB_ROWS)])
    plsc.subcore_barrier()

    def chunk(g, carry):
        r0 = c * 3200 + s * 200 + g * 4
        pltpu.sync_copy(rows_hbm.at[pl.ds(r0, 4), :], rbuf)
        pltpu.sync_copy(cols_hbm.at[pl.ds(r0, 4), :], cbuf)
        for jj in range(4):
            def vec(k, carry2):
                rv = rbuf[jj, pl.ds(k * 16, 16)]
                cv = cbuf[jj, pl.ds(k * 16, 16)]
                eq = rv == cv
                cpbuf[jj, pl.ds(k * 16, 16)] = jnp.where(
                    eq, jnp.full((16,), DUMMY, jnp.int32), cv)
                ewbuf[jj, pl.ds(k * 16, 16)] = jnp.where(
                    eq, jnp.zeros((16,), jnp.float32),
                    jnp.ones((16,), jnp.float32))
                return carry2
            lax.fori_loop(0, 8, vec, None)
        pltpu.sync_copy(cpbuf, colp_hbm.at[pl.ds(r0, 4), :])
        for j in range(4):
            pltpu.sync_copy(ewbuf.at[j], deg_acc.at[rbuf.at[j]], add=True)
        return carry

    lax.fori_loop(0, 50, chunk, None)
    plsc.subcore_barrier()
    pltpu.sync_copy(deg_acc.at[pl.ds(s * SUB_ROWS, SUB_ROWS)],
                    deg_hbm.at[c].at[pl.ds(s * SUB_ROWS, SUB_ROWS)])


def _make_agg(C2, base=0, edge_split=False, chrows=8):
    rows_per_sub = (EROWS // 32 if edge_split else EROWS // 16)
    nch = rows_per_sub // chrows
    npairs = nch // 2
    assert nch % 2 == 0

    @functools.partial(
        pl.kernel,
        out_type=jax.ShapeDtypeStruct((2, NPAD, C2), jnp.float32),
        mesh=_mesh,
        scratch_types=[
            pltpu.VMEM((chrows, 128), jnp.int32),
            pltpu.VMEM((chrows, 128), jnp.int32),
            pltpu.VMEM((chrows, 128), jnp.int32),
            pltpu.VMEM((chrows, 128), jnp.int32),
            pltpu.VMEM((chrows, 128, C2), jnp.float32),
            pltpu.VMEM((chrows, 128, C2), jnp.float32),
            pltpu.VMEM_SHARED((NPAD, C2), jnp.float32),
            pltpu.SemaphoreType.DMA,
            pltpu.SemaphoreType.DMA,
            pltpu.SemaphoreType.DMA,
            pltpu.SemaphoreType.DMA,
        ],
        compiler_params=_sc_params,
    )
    def _agg(g_hbm, rows_hbm, colp_hbm, z_hbm, out_hbm, ridx0, cidx0,
             ridx1, cidx1, data0, data1, acc, semg0, semg1, sems0, sems1):
        c = lax.axis_index("c")
        s = lax.axis_index("s")
        pltpu.sync_copy(z_hbm, acc.at[pl.ds(s * SUB_ROWS, SUB_ROWS), :])
        plsc.subcore_barrier()
        if edge_split:
            gplane = g_hbm.at[base]
            ebase = c * (EROWS // 2) + s * rows_per_sub
        else:
            gplane = g_hbm.at[base + c]
            ebase = s * rows_per_sub

        def fire(k, ridx, cidx, data, semg):
            r0 = ebase + k * chrows
            pltpu.sync_copy(rows_hbm.at[pl.ds(r0, chrows), :], ridx)
            pltpu.sync_copy(colp_hbm.at[pl.ds(r0, chrows), :], cidx)
            for j in range(chrows):
                pltpu.async_copy(gplane.at[ridx.at[j]], data.at[j], semg)

        def scat_start(ridx, cidx, data, semg, sems):
            for j in range(chrows):
                pltpu.make_async_copy(gplane.at[ridx.at[j]], data.at[j],
                                      semg).wait()
            for j in range(chrows):
                pltpu.async_copy(data.at[j], acc.at[cidx.at[j]], sems,
                                 add=True)

        def scat_drain(cidx, data, sems):
            for j in range(chrows):
                pltpu.make_async_copy(data.at[j], acc.at[cidx.at[j]],
                                      sems).wait()

        fire(0, ridx0, cidx0, data0, semg0)
        fire(1, ridx1, cidx1, data1, semg1)

        def pair(i, carry):
            scat_start(ridx0, cidx0, data0, semg0, sems0)
            scat_start(ridx1, cidx1, data1, semg1, sems1)

            @pl.when(i < npairs - 1)
            def _():
                scat_drain(cidx0, data0, sems0)
                fire(2 * i + 2, ridx0, cidx0, data0, semg0)
                scat_drain(cidx1, data1, sems1)
                fire(2 * i + 3, ridx1, cidx1, data1, semg1)

            return carry

        lax.fori_loop(0, npairs, pair, None)
        scat_drain(cidx0, data0, sems0)
        scat_drain(cidx1, data1, sems1)
        plsc.subcore_barrier()
        pltpu.sync_copy(acc.at[pl.ds(s * SUB_ROWS, SUB_ROWS), :],
                        out_hbm.at[c].at[pl.ds(s * SUB_ROWS, SUB_ROWS), :])

    return _agg


_agg16 = _make_agg(16)
_agg16a = _make_agg(16, base=0)
_agg16b = _make_agg(16, base=2)
_agg_l3 = _make_agg(16, base=0, edge_split=True, chrows=4)


def _dis_of(deg_ref):
    return lax.rsqrt(deg_ref[0, :] + deg_ref[1, :] + 1.0)


def _k1_body(deg_ref, x_ref, w_ref, out_ref):
    dis = _dis_of(deg_ref)
    h = jnp.dot(x_ref[...], w_ref[...], preferred_element_type=jnp.float32)
    g = h * dis[:, None]
    out_ref[0] = g[:, :16]
    out_ref[1] = g[:, 16:]


_k1 = pl.pallas_call(
    _k1_body,
    grid=(GRID,),
    in_specs=[
        pl.BlockSpec((2, NODE_BLK), lambda i: (0, i)),
        pl.BlockSpec((NODE_BLK, 128), lambda i: (i, 0)),
        pl.BlockSpec((128, 32), lambda i: (0, 0)),
    ],
    out_specs=pl.BlockSpec((2, NODE_BLK, 16), lambda i: (0, i, 0)),
    out_shape=jax.ShapeDtypeStruct((2, NPAD, 16), jnp.float32),
)


def _make_mid(Cin, Cout, n_acc, in_planes, out_planes):
    Pin = Cin // in_planes
    Pout = Cout // out_planes

    def body(deg_ref, *refs):
        acc_refs = refs[:n_acc]
        g_ref, w_ref, b_ref, out_ref = refs[n_acc:]
        dis = _dis_of(deg_ref)
        accs = [acc_refs[q // 2][q % 2] for q in range(in_planes)]
        z = jnp.concatenate(
            [accs[q] + g_ref[q] for q in range(in_planes)], axis=1)
        h = jax.nn.relu(z * dis[:, None] + b_ref[...])
        h2 = jnp.dot(h, w_ref[...], preferred_element_type=jnp.float32)
        g2 = h2 * dis[:, None]
        for q in range(out_planes):
            out_ref[q] = g2[:, q * Pout:(q + 1) * Pout]

    return pl.pallas_call(
        body,
        grid=(GRID,),
        in_specs=[
            pl.BlockSpec((2, NODE_BLK), lambda i: (0, i)),
        ] + [
            pl.BlockSpec((2, NODE_BLK, Pin), lambda i: (0, i, 0))
            for _ in range(n_acc)
        ] + [
            pl.BlockSpec((in_planes, NODE_BLK, Pin), lambda i: (0, i, 0)),
            pl.BlockSpec((Cin, Cout), lambda i: (0, 0)),
            pl.BlockSpec((1, Cin), lambda i: (0, 0)),
        ],
        out_specs=pl.BlockSpec((out_planes, NODE_BLK, Pout),
                               lambda i: (0, i, 0)),
        out_shape=jax.ShapeDtypeStruct((out_planes, NPAD, Pout), jnp.float32),
    )


_mid12 = _make_mid(32, 64, 1, 2, 4)
_mid23 = _make_mid(64, 16, 2, 4, 1)


def _k7_body(deg_ref, acc_ref, g_ref, wl_ref, b_ref, bl_ref, out_ref):
    i = pl.program_id(0)
    dis = _dis_of(deg_ref)
    z = acc_ref[0] + acc_ref[1] + g_ref[0]
    h = jax.nn.relu(z * dis[:, None] + b_ref[...])
    p = jnp.sum(h * wl_ref[...])

    @pl.when(i == 0)
    def _init():
        out_ref[...] = jnp.zeros((1, 1), jnp.float32)

    out_ref[...] += jnp.full((1, 1), p, jnp.float32)

    @pl.when(i == GRID - 1)
    def _fin():
        out_ref[...] = jax.nn.sigmoid(out_ref[...] + bl_ref[...])


_k7 = pl.pallas_call(
    _k7_body,
    grid=(GRID,),
    in_specs=[
        pl.BlockSpec((2, NODE_BLK), lambda i: (0, i)),
        pl.BlockSpec((2, NODE_BLK, 16), lambda i: (0, i, 0)),
        pl.BlockSpec((1, NODE_BLK, 16), lambda i: (0, i, 0)),
        pl.BlockSpec((NODE_BLK, 16), lambda i: (i, 0)),
        pl.BlockSpec((1, 16), lambda i: (0, 0)),
        pl.BlockSpec((1, 1), lambda i: (0, 0)),
    ],
    out_specs=pl.BlockSpec((1, 1), lambda i: (0, 0)),
    out_shape=jax.ShapeDtypeStruct((1, 1), jnp.float32),
)


def kernel(x, edge_index, W1, b1, W2, b2, W3, b3, Wlin, blin):
    x2 = jnp.zeros((NPAD, 128), jnp.float32).at[:N].set(
        x[0].astype(jnp.float32))
    rows = edge_index[0].astype(jnp.int32)
    cols = edge_index[1].astype(jnp.int32)
    rows_p = jnp.zeros((EPAD,), jnp.int32).at[:E].set(rows).reshape(EROWS, 128)
    cols_p = jnp.zeros((EPAD,), jnp.int32).at[:E].set(cols).reshape(EROWS, 128)

    colp, deg = _prep(rows_p, cols_p)
    g1 = _k1(deg, x2, W1)
    acc1 = _agg16(g1, rows_p, colp, jnp.zeros((SUB_ROWS, 16), jnp.float32))
    g2 = _mid12(deg, acc1, g1, W2, b1.reshape(1, -1))
    z16 = jnp.zeros((SUB_ROWS, 16), jnp.float32)
    acc2a = _agg16a(g2, rows_p, colp, z16)
    acc2b = _agg16b(g2, rows_p, colp, z16)
    g3 = _mid23(deg, acc2a, acc2b, g2, W3, b2.reshape(1, -1))
    acc3 = _agg_l3(g3, rows_p, colp, jnp.zeros((SUB_ROWS, 16), jnp.float32))
    wl = jnp.zeros((NPAD, 16), jnp.float32).at[:N].set(Wlin.reshape(N, 16))
    return _k7(deg, acc3, g3, wl, b3.reshape(1, -1), blin.reshape(1, 1))


# speedup vs baseline: 1.0513x; 1.0513x over previous
import functools
import jax
import jax.numpy as jnp
from jax import lax
from jax.experimental import pallas as pl
from jax.experimental.pallas import tpu as pltpu
from jax.experimental.pallas import tpu_sc as plsc

N = 50000
NODE_BLK = 2048
GRID = 25
NPAD = NODE_BLK * GRID
DUMMY = N
E = 800000
EPAD = 819200
EROWS = EPAD // 128
NSUB = 16
SUB_ROWS = NPAD // NSUB

_mesh = plsc.VectorSubcoreMesh(core_axis_name="c", subcore_axis_name="s")
_sc_params = pltpu.CompilerParams(use_tc_tiling_on_sc=False)


@functools.partial(
    pl.kernel,
    out_type=[
        jax.ShapeDtypeStruct((EROWS, 128), jnp.int32),
        jax.ShapeDtypeStruct((2, NPAD), jnp.float32),
    ],
    mesh=_mesh,
    scratch_types=[
        pltpu.VMEM((4, 128), jnp.int32),
        pltpu.VMEM((4, 128), jnp.int32),
        pltpu.VMEM((4, 128), jnp.int32),
        pltpu.VMEM((4, 128), jnp.float32),
        pltpu.VMEM((SUB_ROWS,), jnp.float32),
        pltpu.VMEM_SHARED((NPAD,), jnp.float32),
    ],
    compiler_params=_sc_params,
)
def _prep(rows_hbm, cols_hbm, colp_hbm, deg_hbm, rbuf, cbuf, cpbuf, ewbuf,
          zbuf, deg_acc):
    c = lax.axis_index("c")
    s = lax.axis_index("s")

    def zb(k, carry):
        zbuf[pl.ds(k * 16, 16)] = jnp.zeros((16,), jnp.float32)
        return carry

    lax.fori_loop(0, SUB_ROWS // 16, zb, None)
    pltpu.sync_copy(zbuf, deg_acc.at[pl.ds(s * SUB_ROWS, SUB_ROWS)])
    plsc.subcore_barrier()

    def chunk(g, carry):
        r0 = c * 3200 + s * 200 + g * 4
        pltpu.sync_copy(rows_hbm.at[pl.ds(r0, 4), :], rbuf)
        pltpu.sync_copy(cols_hbm.at[pl.ds(r0, 4), :], cbuf)
        for jj in range(4):
            def vec(k, carry2):
                rv = rbuf[jj, pl.ds(k * 16, 16)]
                cv = cbuf[jj, pl.ds(k * 16, 16)]
                eq = rv == cv
                cpbuf[jj, pl.ds(k * 16, 16)] = jnp.where(
                    eq, jnp.full((16,), DUMMY, jnp.int32), cv)
                ewbuf[jj, pl.ds(k * 16, 16)] = jnp.where(
                    eq, jnp.zeros((16,), jnp.float32),
                    jnp.ones((16,), jnp.float32))
                return carry2
            lax.fori_loop(0, 8, vec, None)
        pltpu.sync_copy(cpbuf, colp_hbm.at[pl.ds(r0, 4), :])
        for j in range(4):
            pltpu.sync_copy(ewbuf.at[j], deg_acc.at[rbuf.at[j]], add=True)
        return carry

    lax.fori_loop(0, 50, chunk, None)
    plsc.subcore_barrier()
    pltpu.sync_copy(deg_acc.at[pl.ds(s * SUB_ROWS, SUB_ROWS)],
                    deg_hbm.at[c].at[pl.ds(s * SUB_ROWS, SUB_ROWS)])


def _make_agg(C2, bases=(0,), edge_split=False, chrows=8):
    rows_per_sub = (EROWS // 32 if edge_split else EROWS // 16)
    nch = rows_per_sub // chrows
    npairs = nch // 2
    assert nch % 2 == 0

    @functools.partial(
        pl.kernel,
        out_type=jax.ShapeDtypeStruct((2 * len(bases), NPAD, C2),
                                      jnp.float32),
        mesh=_mesh,
        scratch_types=[
            pltpu.VMEM((chrows, 128), jnp.int32),
            pltpu.VMEM((chrows, 128), jnp.int32),
            pltpu.VMEM((chrows, 128), jnp.int32),
            pltpu.VMEM((chrows, 128), jnp.int32),
            pltpu.VMEM((chrows, 128, C2), jnp.float32),
            pltpu.VMEM((chrows, 128, C2), jnp.float32),
            pltpu.VMEM_SHARED((NPAD, C2), jnp.float32),
            pltpu.SemaphoreType.DMA,
            pltpu.SemaphoreType.DMA,
        ],
        compiler_params=_sc_params,
    )
    def _agg(g_hbm, rows_hbm, colp_hbm, z_hbm, out_hbm, ridx0, cidx0,
             ridx1, cidx1, data0, data1, acc, sem0, sem1):
        c = lax.axis_index("c")
        s = lax.axis_index("s")
        if edge_split:
            ebase = c * (EROWS // 2) + s * rows_per_sub
        else:
            ebase = s * rows_per_sub

        for base in bases:
            gplane = g_hbm.at[base] if edge_split else g_hbm.at[base + c]
            pltpu.sync_copy(z_hbm, acc.at[pl.ds(s * SUB_ROWS, SUB_ROWS), :])
            plsc.subcore_barrier()

            def fire(k, ridx, cidx, data, sem):
                r0 = ebase + k * chrows
                pltpu.sync_copy(rows_hbm.at[pl.ds(r0, chrows), :], ridx)
                pltpu.sync_copy(colp_hbm.at[pl.ds(r0, chrows), :], cidx)
                for j in range(chrows):
                    pltpu.async_copy(gplane.at[ridx.at[j]], data.at[j], sem)

            def drain_scatter(ridx, cidx, data, sem):
                for j in range(chrows):
                    pltpu.make_async_copy(gplane.at[ridx.at[j]], data.at[j],
                                          sem).wait()
                for j in range(chrows):
                    pltpu.sync_copy(data.at[j], acc.at[cidx.at[j]], add=True)

            fire(0, ridx0, cidx0, data0, sem0)

            def pair(i, carry):
                fire(2 * i + 1, ridx1, cidx1, data1, sem1)
                drain_scatter(ridx0, cidx0, data0, sem0)

                @pl.when(i < npairs - 1)
                def _():
                    fire(2 * i + 2, ridx0, cidx0, data0, sem0)

                drain_scatter(ridx1, cidx1, data1, sem1)
                return carry

            lax.fori_loop(0, npairs, pair, None)
            plsc.subcore_barrier()
            pltpu.sync_copy(acc.at[pl.ds(s * SUB_ROWS, SUB_ROWS), :],
                            out_hbm.at[base + c].at[
                                pl.ds(s * SUB_ROWS, SUB_ROWS), :])
            plsc.subcore_barrier()

    return _agg


_agg16 = _make_agg(16)
_agg_l2 = _make_agg(16, bases=(0, 2))
_agg_l3 = _make_agg(16, bases=(0,), edge_split=True, chrows=4)


def _dis_of(deg_ref):
    return lax.rsqrt(deg_ref[0, :] + deg_ref[1, :] + 1.0)


def _k1_body(deg_ref, x_ref, w_ref, out_ref):
    dis = _dis_of(deg_ref)
    h = jnp.dot(x_ref[...], w_ref[...], preferred_element_type=jnp.float32)
    g = h * dis[:, None]
    out_ref[0] = g[:, :16]
    out_ref[1] = g[:, 16:]


_k1 = pl.pallas_call(
    _k1_body,
    grid=(GRID,),
    in_specs=[
        pl.BlockSpec((2, NODE_BLK), lambda i: (0, i)),
        pl.BlockSpec((NODE_BLK, 128), lambda i: (i, 0)),
        pl.BlockSpec((128, 32), lambda i: (0, 0)),
    ],
    out_specs=pl.BlockSpec((2, NODE_BLK, 16), lambda i: (0, i, 0)),
    out_shape=jax.ShapeDtypeStruct((2, NPAD, 16), jnp.float32),
)


def _make_mid(Cin, Cout, n_acc, in_planes, out_planes):
    Pin = Cin // in_planes
    Pout = Cout // out_planes

    ppa = in_planes // n_acc

    def body(deg_ref, *refs):
        acc_refs = refs[:n_acc]
        g_ref, w_ref, b_ref, out_ref = refs[n_acc:]
        dis = _dis_of(deg_ref)
        accs = [acc_refs[q // ppa][q % ppa] for q in range(in_planes)]
        z = jnp.concatenate(
            [accs[q] + g_ref[q] for q in range(in_planes)], axis=1)
        h = jax.nn.relu(z * dis[:, None] + b_ref[...])
        h2 = jnp.dot(h, w_ref[...], preferred_element_type=jnp.float32)
        g2 = h2 * dis[:, None]
        for q in range(out_planes):
            out_ref[q] = g2[:, q * Pout:(q + 1) * Pout]

    return pl.pallas_call(
        body,
        grid=(GRID,),
        in_specs=[
            pl.BlockSpec((2, NODE_BLK), lambda i: (0, i)),
        ] + [
            pl.BlockSpec((ppa, NODE_BLK, Pin), lambda i: (0, i, 0))
            for _ in range(n_acc)
        ] + [
            pl.BlockSpec((in_planes, NODE_BLK, Pin), lambda i: (0, i, 0)),
            pl.BlockSpec((Cin, Cout), lambda i: (0, 0)),
            pl.BlockSpec((1, Cin), lambda i: (0, 0)),
        ],
        out_specs=pl.BlockSpec((out_planes, NODE_BLK, Pout),
                               lambda i: (0, i, 0)),
        out_shape=jax.ShapeDtypeStruct((out_planes, NPAD, Pout), jnp.float32),
    )


_mid12 = _make_mid(32, 64, 1, 2, 4)
_mid23 = _make_mid(64, 16, 1, 4, 1)


def _k7_body(deg_ref, acc_ref, g_ref, wl_ref, b_ref, bl_ref, out_ref):
    i = pl.program_id(0)
    dis = _dis_of(deg_ref)
    z = acc_ref[0] + acc_ref[1] + g_ref[0]
    h = jax.nn.relu(z * dis[:, None] + b_ref[...])
    rid = lax.broadcasted_iota(jnp.int32, (NODE_BLK, 16), 0) + i * NODE_BLK
    p = jnp.sum(jnp.where(rid < N, h * wl_ref[...], 0.0))

    @pl.when(i == 0)
    def _init():
        out_ref[...] = jnp.zeros((1, 1), jnp.float32)

    out_ref[...] += jnp.full((1, 1), p, jnp.float32)

    @pl.when(i == GRID - 1)
    def _fin():
        out_ref[...] = jax.nn.sigmoid(out_ref[...] + bl_ref[...])


_k7 = pl.pallas_call(
    _k7_body,
    grid=(GRID,),
    in_specs=[
        pl.BlockSpec((2, NODE_BLK), lambda i: (0, i)),
        pl.BlockSpec((2, NODE_BLK, 16), lambda i: (0, i, 0)),
        pl.BlockSpec((1, NODE_BLK, 16), lambda i: (0, i, 0)),
        pl.BlockSpec((NODE_BLK, 16), lambda i: (i, 0)),
        pl.BlockSpec((1, 16), lambda i: (0, 0)),
        pl.BlockSpec((1, 1), lambda i: (0, 0)),
    ],
    out_specs=pl.BlockSpec((1, 1), lambda i: (0, 0)),
    out_shape=jax.ShapeDtypeStruct((1, 1), jnp.float32),
)


def kernel(x, edge_index, W1, b1, W2, b2, W3, b3, Wlin, blin):
    x2 = x.reshape(N, 128).astype(jnp.float32)
    rows = edge_index[0].astype(jnp.int32)
    cols = edge_index[1].astype(jnp.int32)
    rows_p = jnp.zeros((EPAD,), jnp.int32).at[:E].set(rows).reshape(EROWS, 128)
    cols_p = jnp.zeros((EPAD,), jnp.int32).at[:E].set(cols).reshape(EROWS, 128)
    z16 = jnp.zeros((SUB_ROWS, 16), jnp.float32)

    colp, deg = _prep(rows_p, cols_p)
    g1 = _k1(deg, x2, W1)
    acc1 = _agg16(g1, rows_p, colp, z16)
    g2 = _mid12(deg, acc1, g1, W2, b1.reshape(1, -1))
    acc2 = _agg_l2(g2, rows_p, colp, z16)
    g3 = _mid23(deg, acc2, g2, W3, b2.reshape(1, -1))
    acc3 = _agg_l3(g3, rows_p, colp, z16)
    wl = Wlin.reshape(N, 16)
    return _k7(deg, acc3, g3, wl, b3.reshape(1, -1), blin.reshape(1, 1))


# speedup vs baseline: 1.0621x; 1.0104x over previous
import functools
import jax
import jax.numpy as jnp
from jax import lax
from jax.experimental import pallas as pl
from jax.experimental.pallas import tpu as pltpu
from jax.experimental.pallas import tpu_sc as plsc

N = 50000
NODE_BLK = 2048
GRID = 25
NPAD = NODE_BLK * GRID
DUMMY = N
E = 800000
EPAD = 819200
EROWS = EPAD // 128
NSUB = 16
SUB_ROWS = NPAD // NSUB

_mesh = plsc.VectorSubcoreMesh(core_axis_name="c", subcore_axis_name="s")
_sc_params = pltpu.CompilerParams(use_tc_tiling_on_sc=False)


@functools.partial(
    pl.kernel,
    out_type=[
        jax.ShapeDtypeStruct((EROWS, 128), jnp.int32),
        jax.ShapeDtypeStruct((2, NPAD), jnp.float32),
    ],
    mesh=_mesh,
    scratch_types=[
        pltpu.VMEM((4, 128), jnp.int32),
        pltpu.VMEM((4, 128), jnp.int32),
        pltpu.VMEM((4, 128), jnp.int32),
        pltpu.VMEM((4, 128), jnp.float32),
        pltpu.VMEM((SUB_ROWS,), jnp.float32),
        pltpu.VMEM_SHARED((NPAD,), jnp.float32),
    ],
    compiler_params=_sc_params,
)
def _prep(rows_hbm, cols_hbm, colp_hbm, deg_hbm, rbuf, cbuf, cpbuf, ewbuf,
          zbuf, deg_acc):
    c = lax.axis_index("c")
    s = lax.axis_index("s")

    def zb(k, carry):
        zbuf[pl.ds(k * 16, 16)] = jnp.zeros((16,), jnp.float32)
        return carry

    lax.fori_loop(0, SUB_ROWS // 16, zb, None)
    pltpu.sync_copy(zbuf, deg_acc.at[pl.ds(s * SUB_ROWS, SUB_ROWS)])
    plsc.subcore_barrier()

    def chunk(g, carry):
        r0 = c * 3200 + s * 200 + g * 4
        pltpu.sync_copy(rows_hbm.at[pl.ds(r0, 4), :], rbuf)
        pltpu.sync_copy(cols_hbm.at[pl.ds(r0, 4), :], cbuf)
        for jj in range(4):
            def vec(k, carry2):
                rv = rbuf[jj, pl.ds(k * 16, 16)]
                cv = cbuf[jj, pl.ds(k * 16, 16)]
                eq = rv == cv
                cpbuf[jj, pl.ds(k * 16, 16)] = jnp.where(
                    eq, jnp.full((16,), DUMMY, jnp.int32), cv)
                ewbuf[jj, pl.ds(k * 16, 16)] = jnp.where(
                    eq, jnp.zeros((16,), jnp.float32),
                    jnp.ones((16,), jnp.float32))
                return carry2
            lax.fori_loop(0, 8, vec, None)
        pltpu.sync_copy(cpbuf, colp_hbm.at[pl.ds(r0, 4), :])
        for j in range(4):
            pltpu.sync_copy(ewbuf.at[j], deg_acc.at[rbuf.at[j]], add=True)
        return carry

    lax.fori_loop(0, 50, chunk, None)
    plsc.subcore_barrier()
    pltpu.sync_copy(deg_acc.at[pl.ds(s * SUB_ROWS, SUB_ROWS)],
                    deg_hbm.at[c].at[pl.ds(s * SUB_ROWS, SUB_ROWS)])


def _make_agg(C2, bases=(0,), edge_split=False, chrows=8):
    rows_per_sub = (EROWS // 32 if edge_split else EROWS // 16)
    nch = rows_per_sub // chrows
    npairs = nch // 2
    assert nch % 2 == 0

    @functools.partial(
        pl.kernel,
        out_type=jax.ShapeDtypeStruct((2 * len(bases), NPAD, C2),
                                      jnp.float32),
        mesh=_mesh,
        scratch_types=[
            pltpu.VMEM((chrows, 128), jnp.int32),
            pltpu.VMEM((chrows, 128), jnp.int32),
            pltpu.VMEM((chrows, 128), jnp.int32),
            pltpu.VMEM((chrows, 128), jnp.int32),
            pltpu.VMEM((chrows, 128, C2), jnp.float32),
            pltpu.VMEM((chrows, 128, C2), jnp.float32),
            pltpu.VMEM_SHARED((NPAD, C2), jnp.float32),
            pltpu.SemaphoreType.DMA,
            pltpu.SemaphoreType.DMA,
        ],
        compiler_params=_sc_params,
    )
    def _agg(g_hbm, rows_hbm, colp_hbm, z_hbm, out_hbm, ridx0, cidx0,
             ridx1, cidx1, data0, data1, acc, sem0, sem1):
        c = lax.axis_index("c")
        s = lax.axis_index("s")
        if edge_split:
            ebase = c * (EROWS // 2) + s * rows_per_sub
        else:
            ebase = s * rows_per_sub

        for base in bases:
            gplane = g_hbm.at[base] if edge_split else g_hbm.at[base + c]
            pltpu.sync_copy(z_hbm, acc.at[pl.ds(s * SUB_ROWS, SUB_ROWS), :])
            plsc.subcore_barrier()

            def fire(k, ridx, cidx, data, sem):
                r0 = ebase + k * chrows
                pltpu.sync_copy(rows_hbm.at[pl.ds(r0, chrows), :], ridx)
                pltpu.sync_copy(colp_hbm.at[pl.ds(r0, chrows), :], cidx)
                for j in range(chrows):
                    pltpu.async_copy(gplane.at[ridx.at[j]], data.at[j], sem)

            def drain_scatter(ridx, cidx, data, sem):
                for j in range(chrows):
                    pltpu.make_async_copy(gplane.at[ridx.at[j]], data.at[j],
                                          sem).wait()
                for j in range(chrows):
                    pltpu.sync_copy(data.at[j], acc.at[cidx.at[j]], add=True)

            fire(0, ridx0, cidx0, data0, sem0)

            def pair(i, carry):
                fire(2 * i + 1, ridx1, cidx1, data1, sem1)
                drain_scatter(ridx0, cidx0, data0, sem0)

                @pl.when(i < npairs - 1)
                def _():
                    fire(2 * i + 2, ridx0, cidx0, data0, sem0)

                drain_scatter(ridx1, cidx1, data1, sem1)
                return carry

            lax.fori_loop(0, npairs, pair, None)
            plsc.subcore_barrier()
            pltpu.sync_copy(acc.at[pl.ds(s * SUB_ROWS, SUB_ROWS), :],
                            out_hbm.at[base + c].at[
                                pl.ds(s * SUB_ROWS, SUB_ROWS), :])
            plsc.subcore_barrier()

    return _agg


_agg16 = _make_agg(16, chrows=10)
_agg_l2 = _make_agg(16, bases=(0, 2), chrows=10)
_agg_l3 = _make_agg(16, bases=(0,), edge_split=True, chrows=4)


def _dis_of(deg_ref):
    return lax.rsqrt(deg_ref[0, :] + deg_ref[1, :] + 1.0)


def _k1_body(deg_ref, x_ref, w_ref, out_ref):
    dis = _dis_of(deg_ref)
    h = jnp.dot(x_ref[...], w_ref[...], preferred_element_type=jnp.float32)
    g = h * dis[:, None]
    out_ref[0] = g[:, :16]
    out_ref[1] = g[:, 16:]


_k1 = pl.pallas_call(
    _k1_body,
    grid=(GRID,),
    in_specs=[
        pl.BlockSpec((2, NODE_BLK), lambda i: (0, i)),
        pl.BlockSpec((NODE_BLK, 128), lambda i: (i, 0)),
        pl.BlockSpec((128, 32), lambda i: (0, 0)),
    ],
    out_specs=pl.BlockSpec((2, NODE_BLK, 16), lambda i: (0, i, 0)),
    out_shape=jax.ShapeDtypeStruct((2, NPAD, 16), jnp.float32),
)


def _make_mid(Cin, Cout, n_acc, in_planes, out_planes):
    Pin = Cin // in_planes
    Pout = Cout // out_planes

    ppa = in_planes // n_acc

    def body(deg_ref, *refs):
        acc_refs = refs[:n_acc]
        g_ref, w_ref, b_ref, out_ref = refs[n_acc:]
        dis = _dis_of(deg_ref)
        accs = [acc_refs[q // ppa][q % ppa] for q in range(in_planes)]
        z = jnp.concatenate(
            [accs[q] + g_ref[q] for q in range(in_planes)], axis=1)
        h = jax.nn.relu(z * dis[:, None] + b_ref[...])
        h2 = jnp.dot(h, w_ref[...], preferred_element_type=jnp.float32)
        g2 = h2 * dis[:, None]
        for q in range(out_planes):
            out_ref[q] = g2[:, q * Pout:(q + 1) * Pout]

    return pl.pallas_call(
        body,
        grid=(GRID,),
        in_specs=[
            pl.BlockSpec((2, NODE_BLK), lambda i: (0, i)),
        ] + [
            pl.BlockSpec((ppa, NODE_BLK, Pin), lambda i: (0, i, 0))
            for _ in range(n_acc)
        ] + [
            pl.BlockSpec((in_planes, NODE_BLK, Pin), lambda i: (0, i, 0)),
            pl.BlockSpec((Cin, Cout), lambda i: (0, 0)),
            pl.BlockSpec((1, Cin), lambda i: (0, 0)),
        ],
        out_specs=pl.BlockSpec((out_planes, NODE_BLK, Pout),
                               lambda i: (0, i, 0)),
        out_shape=jax.ShapeDtypeStruct((out_planes, NPAD, Pout), jnp.float32),
    )


_mid12 = _make_mid(32, 64, 1, 2, 4)
_mid23 = _make_mid(64, 16, 1, 4, 1)


def _k7_body(deg_ref, acc_ref, g_ref, wl_ref, b_ref, bl_ref, out_ref):
    i = pl.program_id(0)
    dis = _dis_of(deg_ref)
    z = acc_ref[0] + acc_ref[1] + g_ref[0]
    h = jax.nn.relu(z * dis[:, None] + b_ref[...])
    rid = lax.broadcasted_iota(jnp.int32, (NODE_BLK, 16), 0) + i * NODE_BLK
    p = jnp.sum(jnp.where(rid < N, h * wl_ref[...], 0.0))

    @pl.when(i == 0)
    def _init():
        out_ref[...] = jnp.zeros((1, 1), jnp.float32)

    out_ref[...] += jnp.full((1, 1), p, jnp.float32)

    @pl.when(i == GRID - 1)
    def _fin():
        out_ref[...] = jax.nn.sigmoid(out_ref[...] + bl_ref[...])


_k7 = pl.pallas_call(
    _k7_body,
    grid=(GRID,),
    in_specs=[
        pl.BlockSpec((2, NODE_BLK), lambda i: (0, i)),
        pl.BlockSpec((2, NODE_BLK, 16), lambda i: (0, i, 0)),
        pl.BlockSpec((1, NODE_BLK, 16), lambda i: (0, i, 0)),
        pl.BlockSpec((NODE_BLK, 16), lambda i: (i, 0)),
        pl.BlockSpec((1, 16), lambda i: (0, 0)),
        pl.BlockSpec((1, 1), lambda i: (0, 0)),
    ],
    out_specs=pl.BlockSpec((1, 1), lambda i: (0, 0)),
    out_shape=jax.ShapeDtypeStruct((1, 1), jnp.float32),
)


def kernel(x, edge_index, W1, b1, W2, b2, W3, b3, Wlin, blin):
    x2 = x.reshape(N, 128).astype(jnp.float32)
    rows = edge_index[0].astype(jnp.int32)
    cols = edge_index[1].astype(jnp.int32)
    rows_p = jnp.zeros((EPAD,), jnp.int32).at[:E].set(rows).reshape(EROWS, 128)
    cols_p = jnp.zeros((EPAD,), jnp.int32).at[:E].set(cols).reshape(EROWS, 128)
    z16 = jnp.zeros((SUB_ROWS, 16), jnp.float32)

    colp, deg = _prep(rows_p, cols_p)
    g1 = _k1(deg, x2, W1)
    acc1 = _agg16(g1, rows_p, colp, z16)
    g2 = _mid12(deg, acc1, g1, W2, b1.reshape(1, -1))
    acc2 = _agg_l2(g2, rows_p, colp, z16)
    g3 = _mid23(deg, acc2, g2, W3, b2.reshape(1, -1))
    acc3 = _agg_l3(g3, rows_p, colp, z16)
    wl = Wlin.reshape(N, 16)
    return _k7(deg, acc3, g3, wl, b3.reshape(1, -1), blin.reshape(1, 1))
